# baseline (speedup 1.0000x reference)
import jax
import jax.numpy as jnp
from jax import lax
from jax.experimental import pallas as pl
from jax.experimental.pallas import tpu as pltpu, tpu_sc as plsc

AW = (10.0, 16.0, 33.0)
AH = (13.0, 30.0, 23.0)

NB = 16
NG = 5776
NCH = 15
NV = 181
SLAB = NV * 16
OUTW = SLAB * NCH


def _sc_body(x_ref, y_ref, in_v, out_v):
    wid = lax.axis_index("s") * 2 + lax.axis_index("c")
    b = wid // 2
    h = wid % 2
    g0 = h * (NG - SLAB)
    for c in range(NCH):
        pltpu.sync_copy(x_ref.at[pl.ds(b * (NCH * NG) + c * NG + g0, SLAB)],
                        in_v.at[pl.ds(c * SLAB, SLAB)])
    iot = lax.iota(jnp.int32, 16)

    def body(j, carry):
        lane = j * 16
        g = g0 + lane + iot
        xo = (g % 76).astype(jnp.float32)
        yo = (g // 76).astype(jnp.float32)
        idxb = (lane + iot) * NCH
        for c in range(NCH):
            a, k = c // 5, c % 5
            v = in_v[pl.ds(c * SLAB + lane, 16)]
            if k in (0, 1, 4):
                s = 1.0 / (1.0 + jnp.exp(-v))
                if k == 0:
                    ov = (s + xo) * 8.0
                elif k == 1:
                    ov = (s + yo) * 8.0
                else:
                    ov = s
            else:
                ov = jnp.exp(v) * (AW[a] if k == 2 else AH[a])
            plsc.store_scatter(out_v, [idxb + c], ov)
        return carry

    lax.fori_loop(0, NV, body, 0)
    pltpu.sync_copy(out_v, y_ref.at[pl.ds(b * (NG * NCH) + g0 * NCH, OUTW)])


def kernel(x, device, anchors_index):
    xf = x.reshape(NB * NCH * NG)
    mesh = plsc.VectorSubcoreMesh(core_axis_name="c", subcore_axis_name="s")
    y = pl.kernel(
        _sc_body,
        out_type=jax.ShapeDtypeStruct((NB * NG * NCH,), jnp.float32),
        mesh=mesh,
        scratch_types=[
            pltpu.VMEM((NCH * SLAB,), jnp.float32),
            pltpu.VMEM((OUTW,), jnp.float32),
        ],
        compiler_params=pltpu.CompilerParams(needs_layout_passes=False),
    )(xf)
    return y.reshape(NB, 76, 76, 3, 5), y.reshape(NB, 17328, 5)


# speedup vs baseline: 3.9974x; 3.9974x over previous
import jax
import jax.numpy as jnp
from jax.experimental import pallas as pl

AW = (10.0, 16.0, 33.0)
AH = (13.0, 30.0, 23.0)

HB = 4
GRID = 19


def _body(x_ref, o_ref):
    i = pl.program_id(0)
    wof = jax.lax.broadcasted_iota(jnp.int32, (16, 76), 1).astype(jnp.float32)
    for hh in range(HB):
        hval = (i * HB + hh).astype(jnp.float32)
        for c in range(15):
            a, k = c // 5, c % 5
            v = x_ref[c, hh]
            if k in (0, 1, 4):
                s = jax.nn.sigmoid(v)
                if k == 0:
                    ov = (s + wof) * 8.0
                elif k == 1:
                    ov = (s + hval) * 8.0
                else:
                    ov = s
            else:
                ov = jnp.exp(v) * (AW[a] if k == 2 else AH[a])
            o_ref[hh, a, k] = ov


def kernel(x, device, anchors_index):
    xt = jnp.transpose(x, (1, 2, 0, 3))
    o1 = pl.pallas_call(
        _body,
        grid=(GRID,),
        in_specs=[pl.BlockSpec((15, HB, 16, 76), lambda i: (0, i, 0, 0))],
        out_specs=pl.BlockSpec((HB, 3, 5, 16, 76), lambda i: (i, 0, 0, 0, 0)),
        out_shape=jax.ShapeDtypeStruct((76, 3, 5, 16, 76), jnp.float32),
    )(xt)
    heat = jnp.transpose(o1, (3, 0, 4, 1, 2))
    xx = heat.reshape(16, 17328, 5)
    return heat, xx
